# baseline (speedup 1.0000x reference)
import functools

import jax
import jax.numpy as jnp
from jax import lax
from jax.experimental import pallas as pl
from jax.experimental.pallas import tpu as pltpu
from jax.experimental.pallas import tpu_sc as plsc

NUM_EMB = 100000
D = 128
SCALE_F = 10.0
B = 4096 * 50
NC, NS, L = 2, 16, 16
NW = NC * NS
B_PER_W = B // NW
CH = 128
NCHUNK = B_PER_W // CH


def _sc_gather_scale(table, idx):
    mesh = plsc.VectorSubcoreMesh(core_axis_name="c", subcore_axis_name="s")

    @functools.partial(
        pl.kernel,
        mesh=mesh,
        out_type=jax.ShapeDtypeStruct((B, D), jnp.float32),
        scratch_types=[
            pltpu.VMEM((B_PER_W,), jnp.int32),
            pltpu.VMEM((CH, D), jnp.float32),
            pltpu.VMEM((CH, D), jnp.float32),
            pltpu.SemaphoreType.DMA,
            pltpu.SemaphoreType.DMA,
        ],
    )
    def k(table_hbm, idx_hbm, out_hbm, idx_v, rows0, rows1, sem0, sem1):
        wid = lax.axis_index("s") * NC + lax.axis_index("c")
        base = wid * B_PER_W

        pltpu.sync_copy(idx_hbm.at[pl.ds(base, B_PER_W)], idx_v)

        rows = (rows0, rows1)
        sems = (sem0, sem1)

        def gather(c, buf, sem):
            pltpu.async_copy(
                table_hbm.at[idx_v.at[pl.ds(c * CH, CH)]], buf, sem)

        gather(0, rows0, sem0)
        gather(1, rows1, sem1)

        def step(c0, _):
            for b in range(2):
                c = c0 + b
                buf, sem = rows[b], sems[b]
                pltpu.make_async_copy(
                    table_hbm.at[idx_v.at[pl.ds(0, CH)]], buf, sem).wait()

                @plsc.parallel_loop(0, CH, unroll=4)
                def _(r):
                    for j in range(D // L):
                        s = pl.ds(j * L, L)
                        buf[r, s] = buf[r, s] * SCALE_F

                pltpu.sync_copy(buf, out_hbm.at[pl.ds(base + c * CH, CH)])

                @pl.when(c + 2 < NCHUNK)
                def _():
                    gather(c + 2, buf, sem)
            return ()

        lax.fori_loop(0, NCHUNK // 2, lambda i, a: step(i * 2, a), ())

    return k(table, idx)


def kernel(x, weight):
    idx = x.reshape(-1).astype(jnp.int32)
    out = _sc_gather_scale(weight, idx)
    return out.reshape(x.shape + (D,))


# speedup vs baseline: 1.7038x; 1.7038x over previous
import functools

import jax
import jax.numpy as jnp
from jax import lax
from jax.experimental import pallas as pl
from jax.experimental.pallas import tpu as pltpu
from jax.experimental.pallas import tpu_sc as plsc

D = 128
S = 50
SCALE_F = 10.0
NB_ROWS = 4096
NC, NS, L = 2, 16, 16
NW = NC * NS
ROWS_PER_W = NB_ROWS // NW
CB = 2
NCHUNK = ROWS_PER_W // CB


def _sc_gather_scale(table, idx2d):
    mesh = plsc.VectorSubcoreMesh(core_axis_name="c", subcore_axis_name="s")

    @functools.partial(
        pl.kernel,
        mesh=mesh,
        out_type=jax.ShapeDtypeStruct((NB_ROWS, S, D), jnp.float32),
        scratch_types=[
            pltpu.VMEM((ROWS_PER_W, S), jnp.int32),
            pltpu.VMEM((CB, S, D), jnp.float32),
            pltpu.VMEM((CB, S, D), jnp.float32),
            pltpu.SemaphoreType.DMA,
            pltpu.SemaphoreType.DMA,
        ],
    )
    def k(table_hbm, idx_hbm, out_hbm, idx_v, rows0, rows1, sem0, sem1):
        wid = lax.axis_index("s") * NC + lax.axis_index("c")
        base = wid * ROWS_PER_W

        pltpu.sync_copy(idx_hbm.at[pl.ds(base, ROWS_PER_W)], idx_v)

        rows = (rows0, rows1)
        sems = (sem0, sem1)

        def gather(c, buf, sem):
            for b2 in range(CB):
                pltpu.async_copy(
                    table_hbm.at[idx_v.at[c * CB + b2]], buf.at[b2], sem)

        gather(0, rows0, sem0)
        gather(1, rows1, sem1)

        def step(c0, _):
            for b in range(2):
                c = c0 + b
                buf, sem = rows[b], sems[b]
                for b2 in range(CB):
                    pltpu.make_async_copy(
                        table_hbm.at[idx_v.at[b2]], buf.at[b2], sem).wait()

                for b2 in range(CB):
                    @plsc.parallel_loop(0, S, unroll=2)
                    def _(r):
                        for j in range(D // L):
                            s = pl.ds(j * L, L)
                            buf[b2, r, s] = buf[b2, r, s] * SCALE_F

                pltpu.sync_copy(buf, out_hbm.at[pl.ds(base + c * CB, CB)])

                @pl.when(c + 2 < NCHUNK)
                def _():
                    gather(c + 2, buf, sem)
            return ()

        lax.fori_loop(0, NCHUNK // 2, lambda i, a: step(i * 2, a), ())

    return k(table, idx2d)


def kernel(x, weight):
    return _sc_gather_scale(weight, x.astype(jnp.int32))


# speedup vs baseline: 2.9877x; 1.7535x over previous
import functools

import jax
import jax.numpy as jnp
from jax import lax
from jax.experimental import pallas as pl
from jax.experimental.pallas import tpu as pltpu
from jax.experimental.pallas import tpu_sc as plsc

D = 128
S = 50
NB = 4096
SCALE_F = 10.0
NC, NS, L = 2, 16, 16
NW = NC * NS
SPAN = S * NB // NW
CH = 128
NCHUNK = SPAN // CH


def _sc_gather_scale(table, idx_flat):
    mesh = plsc.VectorSubcoreMesh(core_axis_name="c", subcore_axis_name="s")

    @functools.partial(
        pl.kernel,
        mesh=mesh,
        out_type=jax.ShapeDtypeStruct((S, NB, D), jnp.float32),
        scratch_types=[
            pltpu.VMEM((SPAN,), jnp.int32),
            pltpu.VMEM((CH, D), jnp.float32),
            pltpu.VMEM((CH, D), jnp.float32),
            pltpu.SemaphoreType.DMA,
            pltpu.SemaphoreType.DMA,
        ],
    )
    def k(table_hbm, idx_hbm, out_hbm, idx_v, rows0, rows1, sem0, sem1):
        wid = lax.axis_index("s") * NC + lax.axis_index("c")
        g_base = wid * SPAN

        pltpu.sync_copy(idx_hbm.at[pl.ds(g_base, SPAN)], idx_v)

        rows = (rows0, rows1)
        sems = (sem0, sem1)

        def gather(c, buf, sem):
            pltpu.async_copy(
                table_hbm.at[idx_v.at[pl.ds(c * CH, CH)]], buf, sem)

        gather(0, rows0, sem0)
        gather(1, rows1, sem1)

        def step(c0, _):
            for b in range(2):
                c = c0 + b
                buf, sem = rows[b], sems[b]
                pltpu.make_async_copy(
                    table_hbm.at[idx_v.at[pl.ds(0, CH)]], buf, sem).wait()

                @plsc.parallel_loop(0, CH, unroll=4)
                def _(r):
                    for j in range(D // L):
                        s = pl.ds(j * L, L)
                        buf[r, s] = buf[r, s] * SCALE_F

                g = g_base + c * CH
                pltpu.sync_copy(buf, out_hbm.at[g // NB, pl.ds(g % NB, CH)])

                @pl.when(c + 2 < NCHUNK)
                def _():
                    gather(c + 2, buf, sem)
            return ()

        lax.fori_loop(0, NCHUNK // 2, lambda i, a: step(i * 2, a), ())

    return k(table, idx_flat)


def kernel(x, weight):
    idx_flat = jnp.transpose(x.astype(jnp.int32)).reshape(-1)
    out_t = _sc_gather_scale(weight, idx_flat)
    return jnp.transpose(out_t, (1, 0, 2))


# speedup vs baseline: 3.1497x; 1.0542x over previous
import functools

import jax
import jax.numpy as jnp
from jax import lax
from jax.experimental import pallas as pl
from jax.experimental.pallas import tpu as pltpu
from jax.experimental.pallas import tpu_sc as plsc

D = 128
S = 50
NB = 4096
SCALE_F = 10.0
NC, NS, L = 2, 16, 16
NW = NC * NS
SPAN = S * NB // NW
CH = 128
NCHUNK = SPAN // CH


def _sc_gather_scale(table, idx_flat):
    mesh = plsc.VectorSubcoreMesh(core_axis_name="c", subcore_axis_name="s")

    @functools.partial(
        pl.kernel,
        mesh=mesh,
        out_type=jax.ShapeDtypeStruct((S, NB, D), jnp.float32),
        scratch_types=[
            pltpu.VMEM((SPAN,), jnp.int32),
            pltpu.VMEM((CH, D), jnp.float32),
            pltpu.VMEM((CH, D), jnp.float32),
            pltpu.VMEM((CH, D), jnp.float32),
            pltpu.VMEM((CH, D), jnp.float32),
            pltpu.SemaphoreType.DMA,
            pltpu.SemaphoreType.DMA,
            pltpu.SemaphoreType.DMA,
            pltpu.SemaphoreType.DMA,
        ],
    )
    def k(table_hbm, idx_hbm, out_hbm, idx_v, g0, g1, o0, o1,
          gsem0, gsem1, osem0, osem1):
        wid = lax.axis_index("s") * NC + lax.axis_index("c")
        g_base = wid * SPAN

        pltpu.sync_copy(idx_hbm.at[pl.ds(g_base, SPAN)], idx_v)

        gbufs, obufs = (g0, g1), (o0, o1)
        gsems, osems = (gsem0, gsem1), (osem0, osem1)

        def gather(c, buf, sem):
            pltpu.async_copy(
                table_hbm.at[idx_v.at[pl.ds(c * CH, CH)]], buf, sem)

        def out_slice(c):
            g = g_base + c * CH
            return out_hbm.at[g // NB, pl.ds(g % NB, CH)]

        gather(0, g0, gsem0)
        gather(1, g1, gsem1)

        def step(c0, _):
            for b in range(2):
                c = c0 + b
                gbuf, obuf = gbufs[b], obufs[b]
                gsem, osem = gsems[b], osems[b]
                pltpu.make_async_copy(
                    table_hbm.at[idx_v.at[pl.ds(0, CH)]], gbuf, gsem).wait()
                @pl.when(c >= 2)
                def _():
                    pltpu.make_async_copy(obuf, out_slice(c), osem).wait()

                @plsc.parallel_loop(0, CH, unroll=4)
                def _(r):
                    for j in range(D // L):
                        s = pl.ds(j * L, L)
                        obuf[r, s] = gbuf[r, s] * SCALE_F

                @pl.when(c + 2 < NCHUNK)
                def _():
                    gather(c + 2, gbuf, gsem)
                pltpu.async_copy(obuf, out_slice(c), osem)
            return ()

        lax.fori_loop(0, NCHUNK // 2, lambda i, a: step(i * 2, a), ())

        for b in range(2):
            pltpu.make_async_copy(
                obufs[b], out_slice(NCHUNK - 2 + b), osems[b]).wait()

    return k(table, idx_flat)


def kernel(x, weight):
    idx_flat = jnp.transpose(x.astype(jnp.int32)).reshape(-1)
    out_t = _sc_gather_scale(weight, idx_flat)
    return jnp.transpose(out_t, (1, 0, 2))


# speedup vs baseline: 3.2031x; 1.0170x over previous
import functools

import jax
import jax.numpy as jnp
from jax import lax
from jax.experimental import pallas as pl
from jax.experimental.pallas import tpu as pltpu
from jax.experimental.pallas import tpu_sc as plsc

D = 128
S = 50
NB = 4096
SCALE_F = 10.0
NC, NS, L = 2, 16, 16
NW = NC * NS
SPAN = S * NB // NW
CH = 128
NCHUNK = SPAN // CH


def _sc_gather_scale(table, idx_flat):
    mesh = plsc.VectorSubcoreMesh(core_axis_name="c", subcore_axis_name="s")

    @functools.partial(
        pl.kernel,
        mesh=mesh,
        out_type=jax.ShapeDtypeStruct((S, NB, D), jnp.float32),
        scratch_types=[
            pltpu.VMEM((SPAN,), jnp.int32),
            pltpu.VMEM((CH, D), jnp.float32),
            pltpu.VMEM((CH, D), jnp.float32),
            pltpu.VMEM((CH, D), jnp.float32),
            pltpu.VMEM((CH, D), jnp.float32),
            pltpu.VMEM((CH, D), jnp.float32),
            pltpu.VMEM((CH, D), jnp.float32),
            pltpu.SemaphoreType.DMA,
            pltpu.SemaphoreType.DMA,
            pltpu.SemaphoreType.DMA,
            pltpu.SemaphoreType.DMA,
            pltpu.SemaphoreType.DMA,
            pltpu.SemaphoreType.DMA,
        ],
    )
    def k(table_hbm, idx_hbm, out_hbm, idx_v, g0, g1, g2, o0, o1, o2,
          gsem0, gsem1, gsem2, osem0, osem1, osem2):
        wid = lax.axis_index("s") * NC + lax.axis_index("c")
        g_base = wid * SPAN

        pltpu.sync_copy(idx_hbm.at[pl.ds(g_base, SPAN)], idx_v)

        gbufs, obufs = (g0, g1, g2), (o0, o1, o2)
        gsems, osems = (gsem0, gsem1, gsem2), (osem0, osem1, osem2)

        def gather(c, buf, sem):
            pltpu.async_copy(
                table_hbm.at[idx_v.at[pl.ds(c * CH, CH)]], buf, sem)

        def out_slice(c):
            g = g_base + c * CH
            return out_hbm.at[g // NB, pl.ds(g % NB, CH)]

        gather(0, g0, gsem0)
        gather(1, g1, gsem1)

        NGRP = (NCHUNK + 2) // 3

        def step(c0, _):
            for b in range(3):
                c = c0 + b
                gbuf, obuf = gbufs[b], obufs[b]
                gsem, osem = gsems[b], osems[b]

                @pl.when(c < NCHUNK)
                def _():
                    pltpu.make_async_copy(
                        table_hbm.at[idx_v.at[pl.ds(0, CH)]], gbuf,
                        gsem).wait()

                    @pl.when(c + 2 < NCHUNK)
                    def _():
                        gather(c + 2, gbufs[(b + 2) % 3],
                               gsems[(b + 2) % 3])

                    @pl.when(c >= 3)
                    def _():
                        pltpu.make_async_copy(
                            obuf, out_slice(c), osem).wait()

                    @plsc.parallel_loop(0, CH, unroll=4)
                    def _(r):
                        for j in range(D // L):
                            s = pl.ds(j * L, L)
                            obuf[r, s] = gbuf[r, s] * SCALE_F

                    pltpu.async_copy(obuf, out_slice(c), osem)
            return ()

        lax.fori_loop(0, NGRP, lambda i, a: step(i * 3, a), ())

        for c in range(NCHUNK - 3, NCHUNK):
            pltpu.make_async_copy(
                obufs[c % 3], out_slice(c), osems[c % 3]).wait()

    return k(table, idx_flat)


def kernel(x, weight):
    idx_flat = jnp.transpose(x.astype(jnp.int32)).reshape(-1)
    out_t = _sc_gather_scale(weight, idx_flat)
    return jnp.transpose(out_t, (1, 0, 2))


# speedup vs baseline: 3.2047x; 1.0005x over previous
import functools

import jax
import jax.numpy as jnp
from jax import lax
from jax.experimental import pallas as pl
from jax.experimental.pallas import tpu as pltpu
from jax.experimental.pallas import tpu_sc as plsc

D = 128
S = 50
NB = 4096
SCALE_F = 10.0
NC, NS, L = 2, 16, 16
NW = NC * NS
SPAN = S * NB // NW
CH = 128
NCHUNK = SPAN // CH


def _sc_gather_scale(table, idx_flat):
    mesh = plsc.VectorSubcoreMesh(core_axis_name="c", subcore_axis_name="s")

    @functools.partial(
        pl.kernel,
        mesh=mesh,
        out_type=jax.ShapeDtypeStruct((S, NB, D), jnp.float32),
        scratch_types=[
            pltpu.VMEM((SPAN,), jnp.int32),
            pltpu.VMEM((CH, D), jnp.float32),
            pltpu.VMEM((CH, D), jnp.float32),
            pltpu.VMEM((CH, D), jnp.float32),
            pltpu.VMEM((CH, D), jnp.float32),
            pltpu.VMEM((CH, D), jnp.float32),
            pltpu.VMEM((CH, D), jnp.float32),
            pltpu.SemaphoreType.DMA,
            pltpu.SemaphoreType.DMA,
            pltpu.SemaphoreType.DMA,
            pltpu.SemaphoreType.DMA,
            pltpu.SemaphoreType.DMA,
            pltpu.SemaphoreType.DMA,
        ],
    )
    def k(table_hbm, idx_hbm, out_hbm, idx_v, g0, g1, g2, o0, o1, o2,
          gsem0, gsem1, gsem2, osem0, osem1, osem2):
        wid = lax.axis_index("s") * NC + lax.axis_index("c")
        g_base = wid * SPAN

        pltpu.sync_copy(idx_hbm.at[pl.ds(g_base, SPAN)], idx_v)

        gbufs, obufs = (g0, g1, g2), (o0, o1, o2)
        gsems, osems = (gsem0, gsem1, gsem2), (osem0, osem1, osem2)

        def gather(c, buf, sem):
            pltpu.async_copy(
                table_hbm.at[idx_v.at[pl.ds(c * CH, CH)]], buf, sem)

        def out_slice(c):
            g = g_base + c * CH
            return out_hbm.at[g // NB, pl.ds(g % NB, CH)]

        gather(0, g0, gsem0)
        gather(1, g1, gsem1)

        NGRP = (NCHUNK + 2) // 3

        def step(c0, _):
            for b in range(3):
                c = c0 + b
                gbuf, obuf = gbufs[b], obufs[b]
                gsem, osem = gsems[b], osems[b]

                @pl.when(c < NCHUNK)
                def _():
                    pltpu.make_async_copy(
                        table_hbm.at[idx_v.at[pl.ds(0, CH)]], gbuf,
                        gsem).wait()

                    @pl.when(c + 2 < NCHUNK)
                    def _():
                        gather(c + 2, gbufs[(b + 2) % 3],
                               gsems[(b + 2) % 3])

                    @pl.when(c >= 3)
                    def _():
                        pltpu.make_async_copy(
                            obuf, out_slice(c), osem).wait()

                    @plsc.parallel_loop(0, CH, unroll=8)
                    def _(r):
                        for j in range(D // L):
                            s = pl.ds(j * L, L)
                            obuf[r, s] = gbuf[r, s] * SCALE_F

                    pltpu.async_copy(obuf, out_slice(c), osem)
            return ()

        lax.fori_loop(0, NGRP, lambda i, a: step(i * 3, a), ())

        for c in range(NCHUNK - 3, NCHUNK):
            pltpu.make_async_copy(
                obufs[c % 3], out_slice(c), osems[c % 3]).wait()

    return k(table, idx_flat)


def kernel(x, weight):
    idx_flat = jnp.transpose(x.astype(jnp.int32)).reshape(-1)
    out_t = _sc_gather_scale(weight, idx_flat)
    return jnp.transpose(out_t, (1, 0, 2))
